# baseline (speedup 1.0000x reference)
import functools
import math

import jax
import jax.numpy as jnp
from jax import lax
from jax.experimental import pallas as pl
from jax.experimental.pallas import tpu as pltpu
from jax.experimental.pallas import tpu_sc as plsc

N = 10000
NP = 10240
D = 256
H = 8
DH = 32
K = 17
KP = 32
RB = 128
NEG_INF = -1000000000.0
SENTINEL = 1e30
COORD_PAD = 1e6


def _qkv_body(x_ref, wq_ref, bq_ref, wk_ref, bk_ref, wv_ref, bv_ref,
              q_ref, k_ref, v_ref):
    x = x_ref[...]
    dn = (((1,), (1,)), ((), ()))
    q_ref[...] = lax.dot_general(x, wq_ref[...], dn,
                                 preferred_element_type=jnp.float32) + bq_ref[...]
    k_ref[...] = lax.dot_general(x, wk_ref[...], dn,
                                 preferred_element_type=jnp.float32) + bk_ref[...]
    v_ref[...] = lax.dot_general(x, wv_ref[...], dn,
                                 preferred_element_type=jnp.float32) + bv_ref[...]


def _qkv(xp, Wq, bq, Wk, bk, Wv, bv):
    blk = 256
    grid = (NP // blk,)
    row = lambda i: (i, 0)
    full = lambda i: (0, 0)
    out = jax.ShapeDtypeStruct((NP, D), jnp.float32)
    return pl.pallas_call(
        _qkv_body,
        grid=grid,
        in_specs=[
            pl.BlockSpec((blk, D), row),
            pl.BlockSpec((D, D), full), pl.BlockSpec((1, D), full),
            pl.BlockSpec((D, D), full), pl.BlockSpec((1, D), full),
            pl.BlockSpec((D, D), full), pl.BlockSpec((1, D), full),
        ],
        out_specs=[pl.BlockSpec((blk, D), row)] * 3,
        out_shape=[out, out, out],
    )(xp, Wq, bq.reshape(1, D), Wk, bk.reshape(1, D), Wv, bv.reshape(1, D))


def _knn_body(mrow_ref, mcol_ref, idx_ref, mask_ref, d2_ref):
    xr = mrow_ref[:, 0:1]
    yr = mrow_ref[:, 1:2]
    zr = mrow_ref[:, 2:3]
    xc = mcol_ref[0:1, :]
    yc = mcol_ref[1:2, :]
    zc = mcol_ref[2:3, :]
    sq_r = xr * xr + yr * yr + zr * zr
    sq_c = xc * xc + yc * yc + zc * zc
    cross = lax.dot_general(mrow_ref[:, 0:8], mcol_ref[...],
                            (((1,), (0,)), ((), ())),
                            preferred_element_type=jnp.float32)
    d2 = sq_r + sq_c - 2.0 * cross
    d2_ref[...] = jnp.maximum(d2, 0.0)

    iota = lax.broadcasted_iota(jnp.int32, (RB, NP), 1)
    iota_k = lax.broadcasted_iota(jnp.int32, (RB, KP), 1)

    def ext(i, carry):
        vals, idxs, mprev, iprev = carry
        cur = d2_ref[...]
        rem = (cur > mprev) | ((cur == mprev) & (iota > iprev))
        m = jnp.min(jnp.where(rem, cur, SENTINEL), axis=1, keepdims=True)
        cand = jnp.where(
            (cur == m) & ((m > mprev) | (iota > iprev)),
            iota, jnp.int32(2**30))
        idx = jnp.min(cand, axis=1, keepdims=True)
        vals = jnp.where(iota_k == i, m, vals)
        idxs = jnp.where(iota_k == i, idx, idxs)
        return vals, idxs, m, idx

    vals0 = jnp.full((RB, KP), SENTINEL, jnp.float32)
    idxs0 = jnp.zeros((RB, KP), jnp.int32)
    mprev0 = jnp.full((RB, 1), -1.0, jnp.float32)
    iprev0 = jnp.full((RB, 1), -1, jnp.int32)
    vals, idxs, _, _ = lax.fori_loop(0, K, ext,
                                     (vals0, idxs0, mprev0, iprev0))
    maskf = ((vals <= 100.0) | (vals < 1e-6)).astype(jnp.float32)
    maskf = jnp.concatenate(
        [jnp.ones((RB, 1), jnp.float32), maskf[:, 1:KP]], axis=1)
    idx_ref[...] = idxs
    mask_ref[...] = maskf


def _knn(mrow, mcol):
    rows = mrow.shape[0]
    grid = (rows // RB,)
    return pl.pallas_call(
        _knn_body,
        grid=grid,
        in_specs=[
            pl.BlockSpec((RB, 128), lambda i: (i, 0)),
            pl.BlockSpec((8, NP), lambda i: (0, 0)),
        ],
        out_specs=[
            pl.BlockSpec((RB, KP), lambda i: (i, 0)),
            pl.BlockSpec((RB, KP), lambda i: (i, 0)),
        ],
        out_shape=[
            jax.ShapeDtypeStruct((rows, KP), jnp.int32),
            jax.ShapeDtypeStruct((rows, KP), jnp.float32),
        ],
        scratch_shapes=[pltpu.VMEM((RB, NP), jnp.float32)],
    )(mrow, mcol)


_NC = 2
_NS = 16
_NW = _NC * _NS
_NSPLIT = 4
_HALF = NP // _NSPLIT
_NPW = _HALF // _NW
_SCALE = 1.0 / math.sqrt(DH)


_GATHER_DNUMS = lax.GatherDimensionNumbers(
    offset_dims=(), collapsed_slice_dims=(0,), start_index_map=(0,))


def _splat(vec, lane):
    idx = jnp.full((16, 1), lane, jnp.int32)
    return lax.gather(vec, idx, _GATHER_DNUMS, slice_sizes=(1,),
                      mode=lax.GatherScatterMode.PROMISE_IN_BOUNDS)


_KG = 20
_BN = 4
_NBUF = 2
_NG = _NPW // _BN


def _attn_sc(q_hbm, k_hbm, v_hbm, idxf_hbm, mask_hbm, out_hbm,
             idx_v, mask_v,
             kb0, kb1, vb0, vb1, qb0, qb1, ob0, ob1,
             semk0, semk1, semv0, semv1, semq0, semq1, semo0, semo1):
    kbufs = [kb0, kb1]
    vbufs = [vb0, vb1]
    qbufs = [qb0, qb1]
    obufs = [ob0, ob1]
    semk = [semk0, semk1]
    semv = [semv0, semv1]
    semq = [semq0, semq1]
    semo = [semo0, semo1]

    wid = lax.axis_index("s") * _NC + lax.axis_index("c")
    base = wid * _NPW
    pltpu.sync_copy(idxf_hbm.at[pl.ds(base * _KG, _NPW * _KG)], idx_v)
    pltpu.sync_copy(mask_hbm.at[pl.ds(base, _NPW)], mask_v)
    jv = lax.iota(jnp.int32, 16)

    def issue(g, s):
        ids = idx_v.at[pl.ds(g * _BN * _KG, _BN * _KG)]
        pltpu.async_copy(k_hbm.at[ids], kbufs[s], semk[s])
        pltpu.async_copy(v_hbm.at[ids], vbufs[s], semv[s])
        pltpu.async_copy(q_hbm.at[pl.ds(base + g * _BN, _BN)], qbufs[s], semq[s])

    def compute_node(s, b, i):
        kbuf = kbufs[s]
        vbuf = vbufs[s]
        r0 = b * _KG
        m0 = mask_v[i, pl.ds(0, 16)]
        m1 = mask_v[i, pl.ds(16, 16)]
        qv = [qbufs[s][b, pl.ds(t * 16, 16)] for t in range(16)]
        z16 = jnp.zeros((16,), jnp.float32)

        def lbody(j, acc):
            l0s, l1s = acc
            n0 = []
            n1 = []
            for h in range(H):
                k0 = kbuf[r0 + j, pl.ds(h * DH, 16)]
                k1 = kbuf[r0 + j, pl.ds(h * DH + 16, 16)]
                sc = jnp.sum(qv[2 * h] * k0 + qv[2 * h + 1] * k1)
                n0.append(jnp.where(jv == j, sc, l0s[h]))
                n1.append(jnp.where(jv == j - 16, sc, l1s[h]))
            return (tuple(n0), tuple(n1))

        zz = tuple(z16 for _ in range(H))
        l0s, l1s = lax.fori_loop(0, K, lbody, (zz, zz))

        w0s = []
        w1s = []
        for h in range(H):
            l0 = jnp.where(m0 > 0.0, l0s[h] * _SCALE, NEG_INF)
            l1 = jnp.where(m1 > 0.0, l1s[h] * _SCALE, NEG_INF)
            l1 = jnp.where(jv == 0, l1, NEG_INF)
            mxv = _splat(plsc.cummax(jnp.maximum(l0, l1)), 15)
            e0 = jnp.exp(l0 - mxv)
            e1 = jnp.exp(l1 - mxv)
            e1 = jnp.where(jv == 0, e1, 0.0)
            sv = _splat(plsc.cumsum(e0 + e1), 15)
            w0s.append(e0 / sv)
            w1s.append(e1 / sv)

        zeros = tuple(jnp.zeros((16,), jnp.float32) for _ in range(16))

        def jbody(j, acc):
            ws = [_splat(w0s[h], j) for h in range(H)]
            new = []
            for vreg in range(16):
                vv = vbuf[r0 + j, pl.ds(vreg * 16, 16)]
                new.append(acc[vreg] + ws[vreg // 2] * vv)
            return tuple(new)

        acc = list(lax.fori_loop(0, 16, jbody, zeros))
        ws16 = [_splat(w1s[h], 0) for h in range(H)]
        for vreg in range(16):
            vv = vbuf[r0 + 16, pl.ds(vreg * 16, 16)]
            acc[vreg] = acc[vreg] + ws16[vreg // 2] * vv
        for vreg in range(16):
            obufs[s][b, pl.ds(vreg * 16, 16)] = acc[vreg]

    def group(g, s):
        pltpu.make_async_copy(k_hbm.at[pl.ds(0, _BN * _KG)], kbufs[s], semk[s]).wait()
        pltpu.make_async_copy(v_hbm.at[pl.ds(0, _BN * _KG)], vbufs[s], semv[s]).wait()
        pltpu.make_async_copy(q_hbm.at[pl.ds(0, _BN)], qbufs[s], semq[s]).wait()

        @pl.when(g >= _NBUF)
        def _():
            pltpu.make_async_copy(obufs[s], out_hbm.at[pl.ds(base, _BN)], semo[s]).wait()

        for b in range(_BN):
            compute_node(s, b, g * _BN + b)
        pltpu.async_copy(obufs[s], out_hbm.at[pl.ds(base + g * _BN, _BN)], semo[s])

        @pl.when(g + _NBUF < _NG)
        def _():
            issue(g + _NBUF, s)

    for s in range(_NBUF):
        issue(s, s)

    def body(gg, carry):
        for sb in range(_NBUF):
            group(gg * _NBUF + sb, sb)
        return carry

    lax.fori_loop(0, _NG // _NBUF, body, 0)
    for s in range(_NBUF):
        pltpu.make_async_copy(obufs[s], out_hbm.at[pl.ds(base, _BN)], semo[s]).wait()


def _attention_emu(q, k, v, idx, maskf):
    qq = q.reshape(NP, H, DH)
    kk = k.reshape(NP, H, DH)
    vv = v.reshape(NP, H, DH)
    kn = kk[idx]
    vn = vv[idx]
    logits = jnp.einsum('nhd,nkhd->nhk', qq, kn) * _SCALE
    logits = jnp.where(maskf[:, None, :] > 0, logits, NEG_INF)
    w = jax.nn.softmax(logits, axis=-1)
    w = w * (jnp.arange(KP) < K)[None, None, :]
    return jnp.einsum('nhk,nkhd->nhd', w, vn).reshape(NP, D)


def _attention(q, k, v, idx, maskf):
    mesh = plsc.VectorSubcoreMesh(core_axis_name="c", subcore_axis_name="s")
    fn = functools.partial(
        pl.kernel,
        mesh=mesh,
        compiler_params=pltpu.CompilerParams(needs_layout_passes=False),
        out_type=jax.ShapeDtypeStruct((_HALF, D), jnp.float32),
        scratch_types=(
            [pltpu.VMEM((_NPW * _KG,), jnp.int32),
             pltpu.VMEM((_NPW, KP), jnp.float32)]
            + [pltpu.VMEM((_BN * _KG, D), jnp.float32)] * 4
            + [pltpu.VMEM((_BN, D), jnp.float32)] * 4
            + [pltpu.SemaphoreType.DMA] * 8
        ),
    )(_attn_sc)
    idx_flat = idx[:, :_KG].reshape(-1)
    return fn(q, k, v, idx_flat, maskf)


def _post_body(a_ref, x_ref, wo_ref, bo_ref, lw_ref, lb_ref, y_ref):
    dn = (((1,), (1,)), ((), ()))
    y = lax.dot_general(a_ref[...], wo_ref[...], dn,
                        preferred_element_type=jnp.float32)
    y = y + bo_ref[...] + x_ref[...]
    mu = jnp.mean(y, axis=1, keepdims=True)
    yc = y - mu
    var = jnp.mean(yc * yc, axis=1, keepdims=True)
    y_ref[...] = yc * lax.rsqrt(var + 1e-5) * lw_ref[...] + lb_ref[...]


def _post(attn, xp, Wo, bo, ln_w, ln_b):
    blk = 256
    grid = (NP // blk,)
    row = lambda i: (i, 0)
    full = lambda i: (0, 0)
    return pl.pallas_call(
        _post_body,
        grid=grid,
        in_specs=[
            pl.BlockSpec((blk, D), row),
            pl.BlockSpec((blk, D), row),
            pl.BlockSpec((D, D), full),
            pl.BlockSpec((1, D), full),
            pl.BlockSpec((1, D), full),
            pl.BlockSpec((1, D), full),
        ],
        out_specs=pl.BlockSpec((blk, D), row),
        out_shape=jax.ShapeDtypeStruct((NP, D), jnp.float32),
    )(attn, xp, Wo, bo.reshape(1, D), ln_w.reshape(1, D), ln_b.reshape(1, D))


def kernel(features, means, Wq, bq, Wk, bk, Wv, bv, Wo, bo, ln_w, ln_b):
    xp = jnp.pad(features, ((0, NP - N), (0, 0)))
    mrow = jnp.pad(means, ((0, NP - N), (0, 128 - 3)))
    mcolT = jnp.concatenate([means.T, jnp.zeros((5, N), jnp.float32)], axis=0)
    mcol = jnp.pad(mcolT, ((0, 0), (0, NP - N)), constant_values=COORD_PAD)

    q, k, v = _qkv(xp, Wq, bq, Wk, bk, Wv, bv)
    attns = []
    for c in range(_NSPLIT):
        lo = c * _HALF
        idxc, maskc = _knn(mrow[lo:lo + _HALF], mcol)
        attns.append(_attention(q[lo:lo + _HALF], k, v, idxc, maskc))
    attn = jnp.concatenate(attns, axis=0)
    y = _post(attn, xp, Wo, bo, ln_w, ln_b)
    return y[:N]


# speedup vs baseline: 1.6465x; 1.6465x over previous
import functools
import math

import jax
import jax.numpy as jnp
from jax import lax
from jax.experimental import pallas as pl
from jax.experimental.pallas import tpu as pltpu
from jax.experimental.pallas import tpu_sc as plsc

N = 10000
NP = 10240
D = 256
H = 8
DH = 32
K = 17
KP = 32
RB = 128
NEG_INF = -1000000000.0
SENTINEL = 1e30
COORD_PAD = 1e6


def _qkv_body(x_ref, wq_ref, bq_ref, wk_ref, bk_ref, wv_ref, bv_ref,
              q_ref, k_ref, v_ref):
    x = x_ref[...]
    dn = (((1,), (1,)), ((), ()))
    q_ref[...] = lax.dot_general(x, wq_ref[...], dn,
                                 preferred_element_type=jnp.float32) + bq_ref[...]
    k_ref[...] = lax.dot_general(x, wk_ref[...], dn,
                                 preferred_element_type=jnp.float32) + bk_ref[...]
    v_ref[...] = lax.dot_general(x, wv_ref[...], dn,
                                 preferred_element_type=jnp.float32) + bv_ref[...]


def _qkv(xp, Wq, bq, Wk, bk, Wv, bv):
    blk = 256
    grid = (NP // blk,)
    row = lambda i: (i, 0)
    full = lambda i: (0, 0)
    out = jax.ShapeDtypeStruct((NP, D), jnp.float32)
    return pl.pallas_call(
        _qkv_body,
        grid=grid,
        in_specs=[
            pl.BlockSpec((blk, D), row),
            pl.BlockSpec((D, D), full), pl.BlockSpec((1, D), full),
            pl.BlockSpec((D, D), full), pl.BlockSpec((1, D), full),
            pl.BlockSpec((D, D), full), pl.BlockSpec((1, D), full),
        ],
        out_specs=[pl.BlockSpec((blk, D), row)] * 3,
        out_shape=[out, out, out],
    )(xp, Wq, bq.reshape(1, D), Wk, bk.reshape(1, D), Wv, bv.reshape(1, D))


def _knn_body(mrow_ref, mcol_ref, idx_ref, mask_ref, d2_ref):
    xr = mrow_ref[:, 0:1]
    yr = mrow_ref[:, 1:2]
    zr = mrow_ref[:, 2:3]
    xc = mcol_ref[0:1, :]
    yc = mcol_ref[1:2, :]
    zc = mcol_ref[2:3, :]
    sq_r = xr * xr + yr * yr + zr * zr
    sq_c = xc * xc + yc * yc + zc * zc
    cross = lax.dot_general(mrow_ref[:, 0:8], mcol_ref[...],
                            (((1,), (0,)), ((), ())),
                            preferred_element_type=jnp.float32)
    d2 = sq_r + sq_c - 2.0 * cross
    d2_ref[...] = jnp.maximum(d2, 0.0)

    iota = lax.broadcasted_iota(jnp.int32, (RB, NP), 1)
    iota_k = lax.broadcasted_iota(jnp.int32, (RB, KP), 1)

    def ext(i, carry):
        vals, idxs = carry
        cur = d2_ref[...]
        m = jnp.min(cur, axis=1, keepdims=True)
        cand = jnp.where(cur == m, iota, jnp.int32(2**30))
        idx = jnp.min(cand, axis=1, keepdims=True)
        d2_ref[...] = jnp.where(iota == idx, SENTINEL, cur)
        vals = jnp.where(iota_k == i, m, vals)
        idxs = jnp.where(iota_k == i, idx, idxs)
        return vals, idxs

    vals0 = jnp.full((RB, KP), SENTINEL, jnp.float32)
    idxs0 = jnp.zeros((RB, KP), jnp.int32)
    vals, idxs = lax.fori_loop(0, K, ext, (vals0, idxs0))
    maskf = ((vals <= 100.0) | (vals < 1e-6)).astype(jnp.float32)
    maskf = jnp.concatenate(
        [jnp.ones((RB, 1), jnp.float32), maskf[:, 1:KP]], axis=1)
    idx_ref[...] = idxs
    mask_ref[...] = maskf


def _knn(mrow, mcol):
    rows = mrow.shape[0]
    grid = (rows // RB,)
    return pl.pallas_call(
        _knn_body,
        grid=grid,
        in_specs=[
            pl.BlockSpec((RB, 128), lambda i: (i, 0)),
            pl.BlockSpec((8, NP), lambda i: (0, 0)),
        ],
        out_specs=[
            pl.BlockSpec((RB, KP), lambda i: (i, 0)),
            pl.BlockSpec((RB, KP), lambda i: (i, 0)),
        ],
        out_shape=[
            jax.ShapeDtypeStruct((rows, KP), jnp.int32),
            jax.ShapeDtypeStruct((rows, KP), jnp.float32),
        ],
        scratch_shapes=[pltpu.VMEM((RB, NP), jnp.float32)],
    )(mrow, mcol)


_NC = 2
_NS = 16
_NW = _NC * _NS
_NSPLIT = 8
_HALF = NP // _NSPLIT
_NPW = _HALF // _NW
_SCALE = 1.0 / math.sqrt(DH)


_GATHER_DNUMS = lax.GatherDimensionNumbers(
    offset_dims=(), collapsed_slice_dims=(0,), start_index_map=(0,))


def _splat(vec, lane):
    idx = jnp.full((16, 1), lane, jnp.int32)
    return lax.gather(vec, idx, _GATHER_DNUMS, slice_sizes=(1,),
                      mode=lax.GatherScatterMode.PROMISE_IN_BOUNDS)


_KG = 20
_BN = 4
_NBUF = 2
_NG = _NPW // _BN


def _attn_sc(q_hbm, k_hbm, v_hbm, idxf_hbm, mask_hbm, out_hbm,
             idx_v, mask_v,
             kb0, kb1, vb0, vb1, qb0, qb1, ob0, ob1,
             semk0, semk1, semv0, semv1, semq0, semq1, semo0, semo1):
    kbufs = [kb0, kb1]
    vbufs = [vb0, vb1]
    qbufs = [qb0, qb1]
    obufs = [ob0, ob1]
    semk = [semk0, semk1]
    semv = [semv0, semv1]
    semq = [semq0, semq1]
    semo = [semo0, semo1]

    wid = lax.axis_index("s") * _NC + lax.axis_index("c")
    base = wid * _NPW
    pltpu.sync_copy(idxf_hbm.at[pl.ds(base * _KG, _NPW * _KG)], idx_v)
    pltpu.sync_copy(mask_hbm.at[pl.ds(base, _NPW)], mask_v)
    jv = lax.iota(jnp.int32, 16)

    def issue(g, s):
        ids = idx_v.at[pl.ds(g * _BN * _KG, _BN * _KG)]
        pltpu.async_copy(k_hbm.at[ids], kbufs[s], semk[s])
        pltpu.async_copy(v_hbm.at[ids], vbufs[s], semv[s])
        pltpu.async_copy(q_hbm.at[pl.ds(base + g * _BN, _BN)], qbufs[s], semq[s])

    def compute_node(s, b, i):
        kbuf = kbufs[s]
        vbuf = vbufs[s]
        r0 = b * _KG
        m0 = mask_v[i, pl.ds(0, 16)]
        m1 = mask_v[i, pl.ds(16, 16)]
        qv = [qbufs[s][b, pl.ds(t * 16, 16)] for t in range(16)]
        z16 = jnp.zeros((16,), jnp.float32)

        def lbody(j, acc):
            l0s, l1s = acc
            n0 = []
            n1 = []
            for h in range(H):
                k0 = kbuf[r0 + j, pl.ds(h * DH, 16)]
                k1 = kbuf[r0 + j, pl.ds(h * DH + 16, 16)]
                sc = jnp.sum(qv[2 * h] * k0 + qv[2 * h + 1] * k1)
                n0.append(jnp.where(jv == j, sc, l0s[h]))
                n1.append(jnp.where(jv == j - 16, sc, l1s[h]))
            return (tuple(n0), tuple(n1))

        zz = tuple(z16 for _ in range(H))
        l0s, l1s = lax.fori_loop(0, K, lbody, (zz, zz))

        w0s = []
        w1s = []
        for h in range(H):
            l0 = jnp.where(m0 > 0.0, l0s[h] * _SCALE, NEG_INF)
            l1 = jnp.where(m1 > 0.0, l1s[h] * _SCALE, NEG_INF)
            l1 = jnp.where(jv == 0, l1, NEG_INF)
            mxv = _splat(plsc.cummax(jnp.maximum(l0, l1)), 15)
            e0 = jnp.exp(l0 - mxv)
            e1 = jnp.exp(l1 - mxv)
            e1 = jnp.where(jv == 0, e1, 0.0)
            sv = _splat(plsc.cumsum(e0 + e1), 15)
            w0s.append(e0 / sv)
            w1s.append(e1 / sv)

        zeros = tuple(jnp.zeros((16,), jnp.float32) for _ in range(16))

        def jbody(j, acc):
            ws = [_splat(w0s[h], j) for h in range(H)]
            new = []
            for vreg in range(16):
                vv = vbuf[r0 + j, pl.ds(vreg * 16, 16)]
                new.append(acc[vreg] + ws[vreg // 2] * vv)
            return tuple(new)

        acc = list(lax.fori_loop(0, 16, jbody, zeros))
        ws16 = [_splat(w1s[h], 0) for h in range(H)]
        for vreg in range(16):
            vv = vbuf[r0 + 16, pl.ds(vreg * 16, 16)]
            acc[vreg] = acc[vreg] + ws16[vreg // 2] * vv
        for vreg in range(16):
            obufs[s][b, pl.ds(vreg * 16, 16)] = acc[vreg]

    def group(g, s):
        pltpu.make_async_copy(k_hbm.at[pl.ds(0, _BN * _KG)], kbufs[s], semk[s]).wait()
        pltpu.make_async_copy(v_hbm.at[pl.ds(0, _BN * _KG)], vbufs[s], semv[s]).wait()
        pltpu.make_async_copy(q_hbm.at[pl.ds(0, _BN)], qbufs[s], semq[s]).wait()

        @pl.when(g >= _NBUF)
        def _():
            pltpu.make_async_copy(obufs[s], out_hbm.at[pl.ds(base, _BN)], semo[s]).wait()

        for b in range(_BN):
            compute_node(s, b, g * _BN + b)
        pltpu.async_copy(obufs[s], out_hbm.at[pl.ds(base + g * _BN, _BN)], semo[s])

        @pl.when(g + _NBUF < _NG)
        def _():
            issue(g + _NBUF, s)

    for s in range(_NBUF):
        issue(s, s)

    def body(gg, carry):
        for sb in range(_NBUF):
            group(gg * _NBUF + sb, sb)
        return carry

    lax.fori_loop(0, _NG // _NBUF, body, 0)
    for s in range(_NBUF):
        pltpu.make_async_copy(obufs[s], out_hbm.at[pl.ds(base, _BN)], semo[s]).wait()


def _attention_emu(q, k, v, idx, maskf):
    qq = q.reshape(NP, H, DH)
    kk = k.reshape(NP, H, DH)
    vv = v.reshape(NP, H, DH)
    kn = kk[idx]
    vn = vv[idx]
    logits = jnp.einsum('nhd,nkhd->nhk', qq, kn) * _SCALE
    logits = jnp.where(maskf[:, None, :] > 0, logits, NEG_INF)
    w = jax.nn.softmax(logits, axis=-1)
    w = w * (jnp.arange(KP) < K)[None, None, :]
    return jnp.einsum('nhk,nkhd->nhd', w, vn).reshape(NP, D)


def _attention(q, k, v, idx, maskf):
    mesh = plsc.VectorSubcoreMesh(core_axis_name="c", subcore_axis_name="s")
    fn = functools.partial(
        pl.kernel,
        mesh=mesh,
        compiler_params=pltpu.CompilerParams(needs_layout_passes=False),
        out_type=jax.ShapeDtypeStruct((_HALF, D), jnp.float32),
        scratch_types=(
            [pltpu.VMEM((_NPW * _KG,), jnp.int32),
             pltpu.VMEM((_NPW, KP), jnp.float32)]
            + [pltpu.VMEM((_BN * _KG, D), jnp.float32)] * 4
            + [pltpu.VMEM((_BN, D), jnp.float32)] * 4
            + [pltpu.SemaphoreType.DMA] * 8
        ),
    )(_attn_sc)
    idx_flat = idx[:, :_KG].reshape(-1)
    return fn(q, k, v, idx_flat, maskf)


def _post_body(a_ref, x_ref, wo_ref, bo_ref, lw_ref, lb_ref, y_ref):
    dn = (((1,), (1,)), ((), ()))
    y = lax.dot_general(a_ref[...], wo_ref[...], dn,
                        preferred_element_type=jnp.float32)
    y = y + bo_ref[...] + x_ref[...]
    mu = jnp.mean(y, axis=1, keepdims=True)
    yc = y - mu
    var = jnp.mean(yc * yc, axis=1, keepdims=True)
    y_ref[...] = yc * lax.rsqrt(var + 1e-5) * lw_ref[...] + lb_ref[...]


def _post(attn, xp, Wo, bo, ln_w, ln_b):
    blk = 256
    grid = (NP // blk,)
    row = lambda i: (i, 0)
    full = lambda i: (0, 0)
    return pl.pallas_call(
        _post_body,
        grid=grid,
        in_specs=[
            pl.BlockSpec((blk, D), row),
            pl.BlockSpec((blk, D), row),
            pl.BlockSpec((D, D), full),
            pl.BlockSpec((1, D), full),
            pl.BlockSpec((1, D), full),
            pl.BlockSpec((1, D), full),
        ],
        out_specs=pl.BlockSpec((blk, D), row),
        out_shape=jax.ShapeDtypeStruct((NP, D), jnp.float32),
    )(attn, xp, Wo, bo.reshape(1, D), ln_w.reshape(1, D), ln_b.reshape(1, D))


def kernel(features, means, Wq, bq, Wk, bk, Wv, bv, Wo, bo, ln_w, ln_b):
    xp = jnp.pad(features, ((0, NP - N), (0, 0)))
    mrow = jnp.pad(means, ((0, NP - N), (0, 128 - 3)))
    mcolT = jnp.concatenate([means.T, jnp.zeros((5, N), jnp.float32)], axis=0)
    mcol = jnp.pad(mcolT, ((0, 0), (0, NP - N)), constant_values=COORD_PAD)

    q, k, v = _qkv(xp, Wq, bq, Wk, bk, Wv, bv)
    attns = []
    for c in range(_NSPLIT):
        lo = c * _HALF
        idxc, maskc = _knn(mrow[lo:lo + _HALF], mcol)
        attns.append(_attention(q[lo:lo + _HALF], k, v, idxc, maskc))
    attn = jnp.concatenate(attns, axis=0)
    y = _post(attn, xp, Wo, bo, ln_w, ln_b)
    return y[:N]


# speedup vs baseline: 1.7082x; 1.0375x over previous
import functools
import math

import jax
import jax.numpy as jnp
from jax import lax
from jax.experimental import pallas as pl
from jax.experimental.pallas import tpu as pltpu
from jax.experimental.pallas import tpu_sc as plsc

N = 10000
NP = 10240
D = 256
H = 8
DH = 32
K = 17
KP = 32
RB = 256
NEG_INF = -1000000000.0
SENTINEL = 1e30
COORD_PAD = 1e6


def _qkv_body(x_ref, wq_ref, bq_ref, wk_ref, bk_ref, wv_ref, bv_ref,
              q_ref, k_ref, v_ref):
    x = x_ref[...]
    dn = (((1,), (1,)), ((), ()))
    q_ref[...] = lax.dot_general(x, wq_ref[...], dn,
                                 preferred_element_type=jnp.float32) + bq_ref[...]
    k_ref[...] = lax.dot_general(x, wk_ref[...], dn,
                                 preferred_element_type=jnp.float32) + bk_ref[...]
    v_ref[...] = lax.dot_general(x, wv_ref[...], dn,
                                 preferred_element_type=jnp.float32) + bv_ref[...]


def _qkv(xp, Wq, bq, Wk, bk, Wv, bv):
    blk = 256
    grid = (NP // blk,)
    row = lambda i: (i, 0)
    full = lambda i: (0, 0)
    out = jax.ShapeDtypeStruct((NP, D), jnp.float32)
    return pl.pallas_call(
        _qkv_body,
        grid=grid,
        in_specs=[
            pl.BlockSpec((blk, D), row),
            pl.BlockSpec((D, D), full), pl.BlockSpec((1, D), full),
            pl.BlockSpec((D, D), full), pl.BlockSpec((1, D), full),
            pl.BlockSpec((D, D), full), pl.BlockSpec((1, D), full),
        ],
        out_specs=[pl.BlockSpec((blk, D), row)] * 3,
        out_shape=[out, out, out],
    )(xp, Wq, bq.reshape(1, D), Wk, bk.reshape(1, D), Wv, bv.reshape(1, D))


def _knn_body(mrow_ref, mcol_ref, idx_ref, mask_ref, d2_ref):
    xr = mrow_ref[:, 0:1]
    yr = mrow_ref[:, 1:2]
    zr = mrow_ref[:, 2:3]
    xc = mcol_ref[0:1, :]
    yc = mcol_ref[1:2, :]
    zc = mcol_ref[2:3, :]
    sq_r = xr * xr + yr * yr + zr * zr
    sq_c = xc * xc + yc * yc + zc * zc
    cross = lax.dot_general(mrow_ref[:, 0:8], mcol_ref[...],
                            (((1,), (0,)), ((), ())),
                            preferred_element_type=jnp.float32)
    d2 = sq_r + sq_c - 2.0 * cross
    d2_ref[...] = jnp.maximum(d2, 0.0)

    iota = lax.broadcasted_iota(jnp.int32, (RB, NP), 1)
    iota_k = lax.broadcasted_iota(jnp.int32, (RB, KP), 1)

    def ext(i, carry):
        vals, idxs = carry
        cur = d2_ref[...]
        m = jnp.min(cur, axis=1, keepdims=True)
        cand = jnp.where(cur == m, iota, jnp.int32(2**30))
        idx = jnp.min(cand, axis=1, keepdims=True)
        d2_ref[...] = jnp.where(iota == idx, SENTINEL, cur)
        vals = jnp.where(iota_k == i, m, vals)
        idxs = jnp.where(iota_k == i, idx, idxs)
        return vals, idxs

    vals0 = jnp.full((RB, KP), SENTINEL, jnp.float32)
    idxs0 = jnp.zeros((RB, KP), jnp.int32)
    vals, idxs = lax.fori_loop(0, K, ext, (vals0, idxs0))
    maskf = ((vals <= 100.0) | (vals < 1e-6)).astype(jnp.float32)
    maskf = jnp.concatenate(
        [jnp.ones((RB, 1), jnp.float32), maskf[:, 1:KP]], axis=1)
    idx_ref[...] = idxs
    mask_ref[...] = maskf


def _knn(mrow, mcol):
    rows = mrow.shape[0]
    grid = (rows // RB,)
    return pl.pallas_call(
        _knn_body,
        grid=grid,
        in_specs=[
            pl.BlockSpec((RB, 128), lambda i: (i, 0)),
            pl.BlockSpec((8, NP), lambda i: (0, 0)),
        ],
        out_specs=[
            pl.BlockSpec((RB, KP), lambda i: (i, 0)),
            pl.BlockSpec((RB, KP), lambda i: (i, 0)),
        ],
        out_shape=[
            jax.ShapeDtypeStruct((rows, KP), jnp.int32),
            jax.ShapeDtypeStruct((rows, KP), jnp.float32),
        ],
        scratch_shapes=[pltpu.VMEM((RB, NP), jnp.float32)],
    )(mrow, mcol)


_NC = 2
_NS = 16
_NW = _NC * _NS
_NSPLIT = 8
_HALF = NP // _NSPLIT
_NPW = _HALF // _NW
_SCALE = 1.0 / math.sqrt(DH)


_GATHER_DNUMS = lax.GatherDimensionNumbers(
    offset_dims=(), collapsed_slice_dims=(0,), start_index_map=(0,))


def _splat(vec, lane):
    idx = jnp.full((16, 1), lane, jnp.int32)
    return lax.gather(vec, idx, _GATHER_DNUMS, slice_sizes=(1,),
                      mode=lax.GatherScatterMode.PROMISE_IN_BOUNDS)


_KG = 20
_BN = 4
_NBUF = 2
_NG = _NPW // _BN


def _attn_sc(q_hbm, k_hbm, v_hbm, idxf_hbm, mask_hbm, out_hbm,
             idx_v, mask_v,
             kb0, kb1, vb0, vb1, qb0, qb1, ob0, ob1,
             semk0, semk1, semv0, semv1, semq0, semq1, semo0, semo1):
    kbufs = [kb0, kb1]
    vbufs = [vb0, vb1]
    qbufs = [qb0, qb1]
    obufs = [ob0, ob1]
    semk = [semk0, semk1]
    semv = [semv0, semv1]
    semq = [semq0, semq1]
    semo = [semo0, semo1]

    wid = lax.axis_index("s") * _NC + lax.axis_index("c")
    base = wid * _NPW
    pltpu.sync_copy(idxf_hbm.at[pl.ds(base * _KG, _NPW * _KG)], idx_v)
    pltpu.sync_copy(mask_hbm.at[pl.ds(base, _NPW)], mask_v)
    jv = lax.iota(jnp.int32, 16)

    def issue(g, s):
        ids = idx_v.at[pl.ds(g * _BN * _KG, _BN * _KG)]
        pltpu.async_copy(k_hbm.at[ids], kbufs[s], semk[s])
        pltpu.async_copy(v_hbm.at[ids], vbufs[s], semv[s])
        pltpu.async_copy(q_hbm.at[pl.ds(base + g * _BN, _BN)], qbufs[s], semq[s])

    def compute_node(s, b, i):
        kbuf = kbufs[s]
        vbuf = vbufs[s]
        r0 = b * _KG
        m0 = mask_v[i, pl.ds(0, 16)]
        m1 = mask_v[i, pl.ds(16, 16)]
        qv = [qbufs[s][b, pl.ds(t * 16, 16)] for t in range(16)]
        z16 = jnp.zeros((16,), jnp.float32)

        def lbody(j, acc):
            l0s, l1s = acc
            n0 = []
            n1 = []
            for h in range(H):
                k0 = kbuf[r0 + j, pl.ds(h * DH, 16)]
                k1 = kbuf[r0 + j, pl.ds(h * DH + 16, 16)]
                sc = jnp.sum(qv[2 * h] * k0 + qv[2 * h + 1] * k1)
                n0.append(jnp.where(jv == j, sc, l0s[h]))
                n1.append(jnp.where(jv == j - 16, sc, l1s[h]))
            return (tuple(n0), tuple(n1))

        zz = tuple(z16 for _ in range(H))
        l0s, l1s = lax.fori_loop(0, K, lbody, (zz, zz))

        w0s = []
        w1s = []
        for h in range(H):
            l0 = jnp.where(m0 > 0.0, l0s[h] * _SCALE, NEG_INF)
            l1 = jnp.where(m1 > 0.0, l1s[h] * _SCALE, NEG_INF)
            l1 = jnp.where(jv == 0, l1, NEG_INF)
            mxv = _splat(plsc.cummax(jnp.maximum(l0, l1)), 15)
            e0 = jnp.exp(l0 - mxv)
            e1 = jnp.exp(l1 - mxv)
            e1 = jnp.where(jv == 0, e1, 0.0)
            sv = _splat(plsc.cumsum(e0 + e1), 15)
            w0s.append(e0 / sv)
            w1s.append(e1 / sv)

        zeros = tuple(jnp.zeros((16,), jnp.float32) for _ in range(16))

        def jbody(j, acc):
            ws = [_splat(w0s[h], j) for h in range(H)]
            new = []
            for vreg in range(16):
                vv = vbuf[r0 + j, pl.ds(vreg * 16, 16)]
                new.append(acc[vreg] + ws[vreg // 2] * vv)
            return tuple(new)

        acc = list(lax.fori_loop(0, 16, jbody, zeros))
        ws16 = [_splat(w1s[h], 0) for h in range(H)]
        for vreg in range(16):
            vv = vbuf[r0 + 16, pl.ds(vreg * 16, 16)]
            acc[vreg] = acc[vreg] + ws16[vreg // 2] * vv
        for vreg in range(16):
            obufs[s][b, pl.ds(vreg * 16, 16)] = acc[vreg]

    def group(g, s):
        pltpu.make_async_copy(k_hbm.at[pl.ds(0, _BN * _KG)], kbufs[s], semk[s]).wait()
        pltpu.make_async_copy(v_hbm.at[pl.ds(0, _BN * _KG)], vbufs[s], semv[s]).wait()
        pltpu.make_async_copy(q_hbm.at[pl.ds(0, _BN)], qbufs[s], semq[s]).wait()

        @pl.when(g >= _NBUF)
        def _():
            pltpu.make_async_copy(obufs[s], out_hbm.at[pl.ds(base, _BN)], semo[s]).wait()

        for b in range(_BN):
            compute_node(s, b, g * _BN + b)
        pltpu.async_copy(obufs[s], out_hbm.at[pl.ds(base + g * _BN, _BN)], semo[s])

        @pl.when(g + _NBUF < _NG)
        def _():
            issue(g + _NBUF, s)

    for s in range(_NBUF):
        issue(s, s)

    def body(gg, carry):
        for sb in range(_NBUF):
            group(gg * _NBUF + sb, sb)
        return carry

    lax.fori_loop(0, _NG // _NBUF, body, 0)
    for s in range(_NBUF):
        pltpu.make_async_copy(obufs[s], out_hbm.at[pl.ds(base, _BN)], semo[s]).wait()


def _attention_emu(q, k, v, idx, maskf):
    qq = q.reshape(NP, H, DH)
    kk = k.reshape(NP, H, DH)
    vv = v.reshape(NP, H, DH)
    kn = kk[idx]
    vn = vv[idx]
    logits = jnp.einsum('nhd,nkhd->nhk', qq, kn) * _SCALE
    logits = jnp.where(maskf[:, None, :] > 0, logits, NEG_INF)
    w = jax.nn.softmax(logits, axis=-1)
    w = w * (jnp.arange(KP) < K)[None, None, :]
    return jnp.einsum('nhk,nkhd->nhd', w, vn).reshape(NP, D)


def _attention(q, k, v, idx, maskf):
    mesh = plsc.VectorSubcoreMesh(core_axis_name="c", subcore_axis_name="s")
    fn = functools.partial(
        pl.kernel,
        mesh=mesh,
        compiler_params=pltpu.CompilerParams(needs_layout_passes=False),
        out_type=jax.ShapeDtypeStruct((_HALF, D), jnp.float32),
        scratch_types=(
            [pltpu.VMEM((_NPW * _KG,), jnp.int32),
             pltpu.VMEM((_NPW, KP), jnp.float32)]
            + [pltpu.VMEM((_BN * _KG, D), jnp.float32)] * 4
            + [pltpu.VMEM((_BN, D), jnp.float32)] * 4
            + [pltpu.SemaphoreType.DMA] * 8
        ),
    )(_attn_sc)
    idx_flat = idx[:, :_KG].reshape(-1)
    return fn(q, k, v, idx_flat, maskf)


def _post_body(a_ref, x_ref, wo_ref, bo_ref, lw_ref, lb_ref, y_ref):
    dn = (((1,), (1,)), ((), ()))
    y = lax.dot_general(a_ref[...], wo_ref[...], dn,
                        preferred_element_type=jnp.float32)
    y = y + bo_ref[...] + x_ref[...]
    mu = jnp.mean(y, axis=1, keepdims=True)
    yc = y - mu
    var = jnp.mean(yc * yc, axis=1, keepdims=True)
    y_ref[...] = yc * lax.rsqrt(var + 1e-5) * lw_ref[...] + lb_ref[...]


def _post(attn, xp, Wo, bo, ln_w, ln_b):
    blk = 256
    grid = (NP // blk,)
    row = lambda i: (i, 0)
    full = lambda i: (0, 0)
    return pl.pallas_call(
        _post_body,
        grid=grid,
        in_specs=[
            pl.BlockSpec((blk, D), row),
            pl.BlockSpec((blk, D), row),
            pl.BlockSpec((D, D), full),
            pl.BlockSpec((1, D), full),
            pl.BlockSpec((1, D), full),
            pl.BlockSpec((1, D), full),
        ],
        out_specs=pl.BlockSpec((blk, D), row),
        out_shape=jax.ShapeDtypeStruct((NP, D), jnp.float32),
    )(attn, xp, Wo, bo.reshape(1, D), ln_w.reshape(1, D), ln_b.reshape(1, D))


def kernel(features, means, Wq, bq, Wk, bk, Wv, bv, Wo, bo, ln_w, ln_b):
    xp = jnp.pad(features, ((0, NP - N), (0, 0)))
    mrow = jnp.pad(means, ((0, NP - N), (0, 128 - 3)))
    mcolT = jnp.concatenate([means.T, jnp.zeros((5, N), jnp.float32)], axis=0)
    mcol = jnp.pad(mcolT, ((0, 0), (0, NP - N)), constant_values=COORD_PAD)

    q, k, v = _qkv(xp, Wq, bq, Wk, bk, Wv, bv)
    attns = []
    for c in range(_NSPLIT):
        lo = c * _HALF
        idxc, maskc = _knn(mrow[lo:lo + _HALF], mcol)
        attns.append(_attention(q[lo:lo + _HALF], k, v, idxc, maskc))
    attn = jnp.concatenate(attns, axis=0)
    y = _post(attn, xp, Wo, bo, ln_w, ln_b)
    return y[:N]


# speedup vs baseline: 1.7367x; 1.0167x over previous
import functools
import math

import jax
import jax.numpy as jnp
from jax import lax
from jax.experimental import pallas as pl
from jax.experimental.pallas import tpu as pltpu
from jax.experimental.pallas import tpu_sc as plsc

N = 10000
NP = 10240
D = 256
H = 8
DH = 32
K = 17
KP = 32
RB = 640
NEG_INF = -1000000000.0
SENTINEL = 1e30
COORD_PAD = 1e6


def _qkv_body(x_ref, wq_ref, bq_ref, wk_ref, bk_ref, wv_ref, bv_ref,
              q_ref, k_ref, v_ref):
    x = x_ref[...]
    dn = (((1,), (1,)), ((), ()))
    q_ref[...] = lax.dot_general(x, wq_ref[...], dn,
                                 preferred_element_type=jnp.float32) + bq_ref[...]
    k_ref[...] = lax.dot_general(x, wk_ref[...], dn,
                                 preferred_element_type=jnp.float32) + bk_ref[...]
    v_ref[...] = lax.dot_general(x, wv_ref[...], dn,
                                 preferred_element_type=jnp.float32) + bv_ref[...]


def _qkv(xp, Wq, bq, Wk, bk, Wv, bv):
    blk = 256
    grid = (NP // blk,)
    row = lambda i: (i, 0)
    full = lambda i: (0, 0)
    out = jax.ShapeDtypeStruct((NP, D), jnp.float32)
    return pl.pallas_call(
        _qkv_body,
        grid=grid,
        in_specs=[
            pl.BlockSpec((blk, D), row),
            pl.BlockSpec((D, D), full), pl.BlockSpec((1, D), full),
            pl.BlockSpec((D, D), full), pl.BlockSpec((1, D), full),
            pl.BlockSpec((D, D), full), pl.BlockSpec((1, D), full),
        ],
        out_specs=[pl.BlockSpec((blk, D), row)] * 3,
        out_shape=[out, out, out],
    )(xp, Wq, bq.reshape(1, D), Wk, bk.reshape(1, D), Wv, bv.reshape(1, D))


def _knn_body(mrow_ref, mcol_ref, idx_ref, mask_ref, d2_ref):
    xr = mrow_ref[:, 0:1]
    yr = mrow_ref[:, 1:2]
    zr = mrow_ref[:, 2:3]
    xc = mcol_ref[0:1, :]
    yc = mcol_ref[1:2, :]
    zc = mcol_ref[2:3, :]
    sq_r = xr * xr + yr * yr + zr * zr
    sq_c = xc * xc + yc * yc + zc * zc
    cross = lax.dot_general(mrow_ref[:, 0:8], mcol_ref[...],
                            (((1,), (0,)), ((), ())),
                            preferred_element_type=jnp.float32)
    d2 = sq_r + sq_c - 2.0 * cross
    d2_ref[...] = jnp.maximum(d2, 0.0)

    iota = lax.broadcasted_iota(jnp.int32, (RB, NP), 1)
    iota_k = lax.broadcasted_iota(jnp.int32, (RB, KP), 1)

    def ext(i, carry):
        vals, idxs = carry
        cur = d2_ref[...]
        m = jnp.min(cur, axis=1, keepdims=True)
        cand = jnp.where(cur == m, iota, jnp.int32(2**30))
        idx = jnp.min(cand, axis=1, keepdims=True)
        d2_ref[...] = jnp.where(iota == idx, SENTINEL, cur)
        vals = jnp.where(iota_k == i, m, vals)
        idxs = jnp.where(iota_k == i, idx, idxs)
        return vals, idxs

    vals0 = jnp.full((RB, KP), SENTINEL, jnp.float32)
    idxs0 = jnp.zeros((RB, KP), jnp.int32)
    vals, idxs = lax.fori_loop(0, K, ext, (vals0, idxs0))
    maskf = ((vals <= 100.0) | (vals < 1e-6)).astype(jnp.float32)
    maskf = jnp.concatenate(
        [jnp.ones((RB, 1), jnp.float32), maskf[:, 1:KP]], axis=1)
    idx_ref[...] = idxs
    mask_ref[...] = maskf


def _knn(mrow, mcol):
    rows = mrow.shape[0]
    grid = (rows // RB,)
    return pl.pallas_call(
        _knn_body,
        grid=grid,
        in_specs=[
            pl.BlockSpec((RB, 128), lambda i: (i, 0)),
            pl.BlockSpec((8, NP), lambda i: (0, 0)),
        ],
        out_specs=[
            pl.BlockSpec((RB, KP), lambda i: (i, 0)),
            pl.BlockSpec((RB, KP), lambda i: (i, 0)),
        ],
        out_shape=[
            jax.ShapeDtypeStruct((rows, KP), jnp.int32),
            jax.ShapeDtypeStruct((rows, KP), jnp.float32),
        ],
        scratch_shapes=[pltpu.VMEM((RB, NP), jnp.float32)],
    )(mrow, mcol)


_NC = 2
_NS = 16
_NW = _NC * _NS
_NSPLIT = 8
_HALF = NP // _NSPLIT
_NPW = _HALF // _NW
_SCALE = 1.0 / math.sqrt(DH)


_GATHER_DNUMS = lax.GatherDimensionNumbers(
    offset_dims=(), collapsed_slice_dims=(0,), start_index_map=(0,))


def _splat(vec, lane):
    idx = jnp.full((16, 1), lane, jnp.int32)
    return lax.gather(vec, idx, _GATHER_DNUMS, slice_sizes=(1,),
                      mode=lax.GatherScatterMode.PROMISE_IN_BOUNDS)


_KG = 20
_BN = 4
_NBUF = 2
_NG = _NPW // _BN


def _attn_sc(q_hbm, k_hbm, v_hbm, idxf_hbm, mask_hbm, out_hbm,
             idx_v, mask_v,
             kb0, kb1, vb0, vb1, qb0, qb1, ob0, ob1,
             semk0, semk1, semv0, semv1, semq0, semq1, semo0, semo1):
    kbufs = [kb0, kb1]
    vbufs = [vb0, vb1]
    qbufs = [qb0, qb1]
    obufs = [ob0, ob1]
    semk = [semk0, semk1]
    semv = [semv0, semv1]
    semq = [semq0, semq1]
    semo = [semo0, semo1]

    wid = lax.axis_index("s") * _NC + lax.axis_index("c")
    base = wid * _NPW
    pltpu.sync_copy(idxf_hbm.at[pl.ds(base * _KG, _NPW * _KG)], idx_v)
    pltpu.sync_copy(mask_hbm.at[pl.ds(base, _NPW)], mask_v)
    jv = lax.iota(jnp.int32, 16)

    def issue(g, s):
        ids = idx_v.at[pl.ds(g * _BN * _KG, _BN * _KG)]
        pltpu.async_copy(k_hbm.at[ids], kbufs[s], semk[s])
        pltpu.async_copy(v_hbm.at[ids], vbufs[s], semv[s])
        pltpu.async_copy(q_hbm.at[pl.ds(base + g * _BN, _BN)], qbufs[s], semq[s])

    def compute_node(s, b, i):
        kbuf = kbufs[s]
        vbuf = vbufs[s]
        r0 = b * _KG
        m0 = mask_v[i, pl.ds(0, 16)]
        m1 = mask_v[i, pl.ds(16, 16)]
        qv = [qbufs[s][b, pl.ds(t * 16, 16)] for t in range(16)]
        z16 = jnp.zeros((16,), jnp.float32)

        def lbody(j, acc):
            l0s, l1s = acc
            n0 = []
            n1 = []
            for h in range(H):
                k0 = kbuf[r0 + j, pl.ds(h * DH, 16)]
                k1 = kbuf[r0 + j, pl.ds(h * DH + 16, 16)]
                sc = jnp.sum(qv[2 * h] * k0 + qv[2 * h + 1] * k1)
                n0.append(jnp.where(jv == j, sc, l0s[h]))
                n1.append(jnp.where(jv == j - 16, sc, l1s[h]))
            return (tuple(n0), tuple(n1))

        zz = tuple(z16 for _ in range(H))
        l0s, l1s = lax.fori_loop(0, K, lbody, (zz, zz))

        w0s = []
        w1s = []
        for h in range(H):
            l0 = jnp.where(m0 > 0.0, l0s[h] * _SCALE, NEG_INF)
            l1 = jnp.where(m1 > 0.0, l1s[h] * _SCALE, NEG_INF)
            l1 = jnp.where(jv == 0, l1, NEG_INF)
            mxv = _splat(plsc.cummax(jnp.maximum(l0, l1)), 15)
            e0 = jnp.exp(l0 - mxv)
            e1 = jnp.exp(l1 - mxv)
            e1 = jnp.where(jv == 0, e1, 0.0)
            sv = _splat(plsc.cumsum(e0 + e1), 15)
            w0s.append(e0 / sv)
            w1s.append(e1 / sv)

        zeros = tuple(jnp.zeros((16,), jnp.float32) for _ in range(16))

        def jbody(j, acc):
            ws = [_splat(w0s[h], j) for h in range(H)]
            new = []
            for vreg in range(16):
                vv = vbuf[r0 + j, pl.ds(vreg * 16, 16)]
                new.append(acc[vreg] + ws[vreg // 2] * vv)
            return tuple(new)

        acc = list(lax.fori_loop(0, 16, jbody, zeros))
        ws16 = [_splat(w1s[h], 0) for h in range(H)]
        for vreg in range(16):
            vv = vbuf[r0 + 16, pl.ds(vreg * 16, 16)]
            acc[vreg] = acc[vreg] + ws16[vreg // 2] * vv
        for vreg in range(16):
            obufs[s][b, pl.ds(vreg * 16, 16)] = acc[vreg]

    def group(g, s):
        pltpu.make_async_copy(k_hbm.at[pl.ds(0, _BN * _KG)], kbufs[s], semk[s]).wait()
        pltpu.make_async_copy(v_hbm.at[pl.ds(0, _BN * _KG)], vbufs[s], semv[s]).wait()
        pltpu.make_async_copy(q_hbm.at[pl.ds(0, _BN)], qbufs[s], semq[s]).wait()

        @pl.when(g >= _NBUF)
        def _():
            pltpu.make_async_copy(obufs[s], out_hbm.at[pl.ds(base, _BN)], semo[s]).wait()

        for b in range(_BN):
            compute_node(s, b, g * _BN + b)
        pltpu.async_copy(obufs[s], out_hbm.at[pl.ds(base + g * _BN, _BN)], semo[s])

        @pl.when(g + _NBUF < _NG)
        def _():
            issue(g + _NBUF, s)

    for s in range(_NBUF):
        issue(s, s)

    def body(gg, carry):
        for sb in range(_NBUF):
            group(gg * _NBUF + sb, sb)
        return carry

    lax.fori_loop(0, _NG // _NBUF, body, 0)
    for s in range(_NBUF):
        pltpu.make_async_copy(obufs[s], out_hbm.at[pl.ds(base, _BN)], semo[s]).wait()


def _attention_emu(q, k, v, idx, maskf):
    qq = q.reshape(NP, H, DH)
    kk = k.reshape(NP, H, DH)
    vv = v.reshape(NP, H, DH)
    kn = kk[idx]
    vn = vv[idx]
    logits = jnp.einsum('nhd,nkhd->nhk', qq, kn) * _SCALE
    logits = jnp.where(maskf[:, None, :] > 0, logits, NEG_INF)
    w = jax.nn.softmax(logits, axis=-1)
    w = w * (jnp.arange(KP) < K)[None, None, :]
    return jnp.einsum('nhk,nkhd->nhd', w, vn).reshape(NP, D)


def _attention(q, k, v, idx, maskf):
    mesh = plsc.VectorSubcoreMesh(core_axis_name="c", subcore_axis_name="s")
    fn = functools.partial(
        pl.kernel,
        mesh=mesh,
        compiler_params=pltpu.CompilerParams(needs_layout_passes=False),
        out_type=jax.ShapeDtypeStruct((_HALF, D), jnp.float32),
        scratch_types=(
            [pltpu.VMEM((_NPW * _KG,), jnp.int32),
             pltpu.VMEM((_NPW, KP), jnp.float32)]
            + [pltpu.VMEM((_BN * _KG, D), jnp.float32)] * 4
            + [pltpu.VMEM((_BN, D), jnp.float32)] * 4
            + [pltpu.SemaphoreType.DMA] * 8
        ),
    )(_attn_sc)
    idx_flat = idx[:, :_KG].reshape(-1)
    return fn(q, k, v, idx_flat, maskf)


def _post_body(a_ref, x_ref, wo_ref, bo_ref, lw_ref, lb_ref, y_ref):
    dn = (((1,), (1,)), ((), ()))
    y = lax.dot_general(a_ref[...], wo_ref[...], dn,
                        preferred_element_type=jnp.float32)
    y = y + bo_ref[...] + x_ref[...]
    mu = jnp.mean(y, axis=1, keepdims=True)
    yc = y - mu
    var = jnp.mean(yc * yc, axis=1, keepdims=True)
    y_ref[...] = yc * lax.rsqrt(var + 1e-5) * lw_ref[...] + lb_ref[...]


def _post(attn, xp, Wo, bo, ln_w, ln_b):
    blk = 256
    grid = (NP // blk,)
    row = lambda i: (i, 0)
    full = lambda i: (0, 0)
    return pl.pallas_call(
        _post_body,
        grid=grid,
        in_specs=[
            pl.BlockSpec((blk, D), row),
            pl.BlockSpec((blk, D), row),
            pl.BlockSpec((D, D), full),
            pl.BlockSpec((1, D), full),
            pl.BlockSpec((1, D), full),
            pl.BlockSpec((1, D), full),
        ],
        out_specs=pl.BlockSpec((blk, D), row),
        out_shape=jax.ShapeDtypeStruct((NP, D), jnp.float32),
    )(attn, xp, Wo, bo.reshape(1, D), ln_w.reshape(1, D), ln_b.reshape(1, D))


def kernel(features, means, Wq, bq, Wk, bk, Wv, bv, Wo, bo, ln_w, ln_b):
    xp = jnp.pad(features, ((0, NP - N), (0, 0)))
    mrow = jnp.pad(means, ((0, NP - N), (0, 128 - 3)))
    mcolT = jnp.concatenate([means.T, jnp.zeros((5, N), jnp.float32)], axis=0)
    mcol = jnp.pad(mcolT, ((0, 0), (0, NP - N)), constant_values=COORD_PAD)

    q, k, v = _qkv(xp, Wq, bq, Wk, bk, Wv, bv)
    attns = []
    for c in range(_NSPLIT):
        lo = c * _HALF
        idxc, maskc = _knn(mrow[lo:lo + _HALF], mcol)
        attns.append(_attention(q[lo:lo + _HALF], k, v, idxc, maskc))
    attn = jnp.concatenate(attns, axis=0)
    y = _post(attn, xp, Wo, bo, ln_w, ln_b)
    return y[:N]


# speedup vs baseline: 1.8132x; 1.0440x over previous
import functools
import math

import jax
import jax.numpy as jnp
from jax import lax
from jax.experimental import pallas as pl
from jax.experimental.pallas import tpu as pltpu
from jax.experimental.pallas import tpu_sc as plsc

N = 10000
NP = 10240
D = 256
H = 8
DH = 32
K = 17
KP = 32
RB = 512
NEG_INF = -1000000000.0
SENTINEL = 1e30
COORD_PAD = 1e6


def _qkv_body(x_ref, wq_ref, bq_ref, wk_ref, bk_ref, wv_ref, bv_ref,
              q_ref, k_ref, v_ref):
    x = x_ref[...]
    dn = (((1,), (1,)), ((), ()))
    q_ref[...] = lax.dot_general(x, wq_ref[...], dn,
                                 preferred_element_type=jnp.float32) + bq_ref[...]
    k_ref[...] = lax.dot_general(x, wk_ref[...], dn,
                                 preferred_element_type=jnp.float32) + bk_ref[...]
    v_ref[...] = lax.dot_general(x, wv_ref[...], dn,
                                 preferred_element_type=jnp.float32) + bv_ref[...]


def _qkv(xp, Wq, bq, Wk, bk, Wv, bv):
    blk = 256
    grid = (NP // blk,)
    row = lambda i: (i, 0)
    full = lambda i: (0, 0)
    out = jax.ShapeDtypeStruct((NP, D), jnp.float32)
    return pl.pallas_call(
        _qkv_body,
        grid=grid,
        in_specs=[
            pl.BlockSpec((blk, D), row),
            pl.BlockSpec((D, D), full), pl.BlockSpec((1, D), full),
            pl.BlockSpec((D, D), full), pl.BlockSpec((1, D), full),
            pl.BlockSpec((D, D), full), pl.BlockSpec((1, D), full),
        ],
        out_specs=[pl.BlockSpec((blk, D), row)] * 3,
        out_shape=[out, out, out],
    )(xp, Wq, bq.reshape(1, D), Wk, bk.reshape(1, D), Wv, bv.reshape(1, D))


def _knn_body(mrow_ref, mcol_ref, idx_ref, mask_ref, d2_ref):
    xr = mrow_ref[:, 0:1]
    yr = mrow_ref[:, 1:2]
    zr = mrow_ref[:, 2:3]
    xc = mcol_ref[0:1, :]
    yc = mcol_ref[1:2, :]
    zc = mcol_ref[2:3, :]
    sq_r = xr * xr + yr * yr + zr * zr
    sq_c = xc * xc + yc * yc + zc * zc
    cross = lax.dot_general(mrow_ref[:, 0:8], mcol_ref[...],
                            (((1,), (0,)), ((), ())),
                            preferred_element_type=jnp.float32)
    d2 = sq_r + sq_c - 2.0 * cross
    d2_ref[...] = jnp.maximum(d2, 0.0)

    iota = lax.broadcasted_iota(jnp.int32, (RB, NP), 1)
    iota_k = lax.broadcasted_iota(jnp.int32, (RB, KP), 1)

    def ext(i, carry):
        vals, idxs = carry
        cur = d2_ref[...]
        m = jnp.min(cur, axis=1, keepdims=True)
        cand = jnp.where(cur == m, iota, jnp.int32(2**30))
        idx = jnp.min(cand, axis=1, keepdims=True)
        d2_ref[...] = jnp.where(iota == idx, SENTINEL, cur)
        vals = jnp.where(iota_k == i, m, vals)
        idxs = jnp.where(iota_k == i, idx, idxs)
        return vals, idxs

    vals0 = jnp.full((RB, KP), SENTINEL, jnp.float32)
    idxs0 = jnp.zeros((RB, KP), jnp.int32)
    vals, idxs = lax.fori_loop(0, K, ext, (vals0, idxs0))
    maskf = ((vals <= 100.0) | (vals < 1e-6)).astype(jnp.float32)
    maskf = jnp.concatenate(
        [jnp.ones((RB, 1), jnp.float32), maskf[:, 1:KP]], axis=1)
    idx_ref[...] = idxs
    mask_ref[...] = maskf


def _knn(mrow, mcol):
    rows = mrow.shape[0]
    grid = (rows // RB,)
    return pl.pallas_call(
        _knn_body,
        grid=grid,
        in_specs=[
            pl.BlockSpec((RB, 128), lambda i: (i, 0)),
            pl.BlockSpec((8, NP), lambda i: (0, 0)),
        ],
        out_specs=[
            pl.BlockSpec((RB, KP), lambda i: (i, 0)),
            pl.BlockSpec((RB, KP), lambda i: (i, 0)),
        ],
        out_shape=[
            jax.ShapeDtypeStruct((rows, KP), jnp.int32),
            jax.ShapeDtypeStruct((rows, KP), jnp.float32),
        ],
        scratch_shapes=[pltpu.VMEM((RB, NP), jnp.float32)],
    )(mrow, mcol)


_NC = 2
_NS = 16
_NW = _NC * _NS
_NSPLIT = 10
_HALF = NP // _NSPLIT
_NPW = _HALF // _NW
_SCALE = 1.0 / math.sqrt(DH)


_GATHER_DNUMS = lax.GatherDimensionNumbers(
    offset_dims=(), collapsed_slice_dims=(0,), start_index_map=(0,))


def _splat(vec, lane):
    idx = jnp.full((16, 1), lane, jnp.int32)
    return lax.gather(vec, idx, _GATHER_DNUMS, slice_sizes=(1,),
                      mode=lax.GatherScatterMode.PROMISE_IN_BOUNDS)


_KG = 18
_BN = 4
_NBUF = 2
_NG = _NPW // _BN


def _attn_sc(q_hbm, k_hbm, v_hbm, idxf_hbm, mask_hbm, out_hbm,
             idx_v, mask_v,
             kb0, kb1, vb0, vb1, qb0, qb1, ob0, ob1,
             semk0, semk1, semv0, semv1, semq0, semq1, semo0, semo1):
    kbufs = [kb0, kb1]
    vbufs = [vb0, vb1]
    qbufs = [qb0, qb1]
    obufs = [ob0, ob1]
    semk = [semk0, semk1]
    semv = [semv0, semv1]
    semq = [semq0, semq1]
    semo = [semo0, semo1]

    wid = lax.axis_index("s") * _NC + lax.axis_index("c")
    base = wid * _NPW
    pltpu.sync_copy(idxf_hbm.at[pl.ds(base * _KG, _NPW * _KG)], idx_v)
    pltpu.sync_copy(mask_hbm.at[pl.ds(base, _NPW)], mask_v)
    jv = lax.iota(jnp.int32, 16)

    def issue(g, s):
        ids = idx_v.at[pl.ds(g * _BN * _KG, _BN * _KG)]
        pltpu.async_copy(k_hbm.at[ids], kbufs[s], semk[s])
        pltpu.async_copy(v_hbm.at[ids], vbufs[s], semv[s])
        pltpu.async_copy(q_hbm.at[pl.ds(base + g * _BN, _BN)], qbufs[s], semq[s])

    def compute_node(s, b, i):
        kbuf = kbufs[s]
        vbuf = vbufs[s]
        r0 = b * _KG
        m0 = mask_v[i, pl.ds(0, 16)]
        m1 = mask_v[i, pl.ds(16, 16)]
        qv = [qbufs[s][b, pl.ds(t * 16, 16)] for t in range(16)]
        z16 = jnp.zeros((16,), jnp.float32)

        def lbody(j, acc):
            l0s, l1s = acc
            n0 = []
            n1 = []
            for h in range(H):
                k0 = kbuf[r0 + j, pl.ds(h * DH, 16)]
                k1 = kbuf[r0 + j, pl.ds(h * DH + 16, 16)]
                sc = jnp.sum(qv[2 * h] * k0 + qv[2 * h + 1] * k1)
                n0.append(jnp.where(jv == j, sc, l0s[h]))
                n1.append(jnp.where(jv == j - 16, sc, l1s[h]))
            return (tuple(n0), tuple(n1))

        zz = tuple(z16 for _ in range(H))
        l0s, l1s = lax.fori_loop(0, K, lbody, (zz, zz))

        w0s = []
        w1s = []
        for h in range(H):
            l0 = jnp.where(m0 > 0.0, l0s[h] * _SCALE, NEG_INF)
            l1 = jnp.where(m1 > 0.0, l1s[h] * _SCALE, NEG_INF)
            l1 = jnp.where(jv == 0, l1, NEG_INF)
            mxv = _splat(plsc.cummax(jnp.maximum(l0, l1)), 15)
            e0 = jnp.exp(l0 - mxv)
            e1 = jnp.exp(l1 - mxv)
            e1 = jnp.where(jv == 0, e1, 0.0)
            sv = _splat(plsc.cumsum(e0 + e1), 15)
            w0s.append(e0 / sv)
            w1s.append(e1 / sv)

        zeros = tuple(jnp.zeros((16,), jnp.float32) for _ in range(16))

        def jbody(j, acc):
            ws = [_splat(w0s[h], j) for h in range(H)]
            new = []
            for vreg in range(16):
                vv = vbuf[r0 + j, pl.ds(vreg * 16, 16)]
                new.append(acc[vreg] + ws[vreg // 2] * vv)
            return tuple(new)

        acc = list(lax.fori_loop(0, 16, jbody, zeros))
        ws16 = [_splat(w1s[h], 0) for h in range(H)]
        for vreg in range(16):
            vv = vbuf[r0 + 16, pl.ds(vreg * 16, 16)]
            acc[vreg] = acc[vreg] + ws16[vreg // 2] * vv
        for vreg in range(16):
            obufs[s][b, pl.ds(vreg * 16, 16)] = acc[vreg]

    def group(g, s):
        pltpu.make_async_copy(k_hbm.at[pl.ds(0, _BN * _KG)], kbufs[s], semk[s]).wait()
        pltpu.make_async_copy(v_hbm.at[pl.ds(0, _BN * _KG)], vbufs[s], semv[s]).wait()
        pltpu.make_async_copy(q_hbm.at[pl.ds(0, _BN)], qbufs[s], semq[s]).wait()

        @pl.when(g >= _NBUF)
        def _():
            pltpu.make_async_copy(obufs[s], out_hbm.at[pl.ds(base, _BN)], semo[s]).wait()

        for b in range(_BN):
            compute_node(s, b, g * _BN + b)
        pltpu.async_copy(obufs[s], out_hbm.at[pl.ds(base + g * _BN, _BN)], semo[s])

        @pl.when(g + _NBUF < _NG)
        def _():
            issue(g + _NBUF, s)

    for s in range(_NBUF):
        issue(s, s)

    def body(gg, carry):
        for sb in range(_NBUF):
            group(gg * _NBUF + sb, sb)
        return carry

    lax.fori_loop(0, _NG // _NBUF, body, 0)
    for s in range(_NBUF):
        pltpu.make_async_copy(obufs[s], out_hbm.at[pl.ds(base, _BN)], semo[s]).wait()


def _attention_emu(q, k, v, idx, maskf):
    qq = q.reshape(NP, H, DH)
    kk = k.reshape(NP, H, DH)
    vv = v.reshape(NP, H, DH)
    kn = kk[idx]
    vn = vv[idx]
    logits = jnp.einsum('nhd,nkhd->nhk', qq, kn) * _SCALE
    logits = jnp.where(maskf[:, None, :] > 0, logits, NEG_INF)
    w = jax.nn.softmax(logits, axis=-1)
    w = w * (jnp.arange(KP) < K)[None, None, :]
    return jnp.einsum('nhk,nkhd->nhd', w, vn).reshape(NP, D)


def _attention(q, k, v, idx, maskf):
    mesh = plsc.VectorSubcoreMesh(core_axis_name="c", subcore_axis_name="s")
    fn = functools.partial(
        pl.kernel,
        mesh=mesh,
        compiler_params=pltpu.CompilerParams(needs_layout_passes=False),
        out_type=jax.ShapeDtypeStruct((_HALF, D), jnp.float32),
        scratch_types=(
            [pltpu.VMEM((_NPW * _KG,), jnp.int32),
             pltpu.VMEM((_NPW, KP), jnp.float32)]
            + [pltpu.VMEM((_BN * _KG, D), jnp.float32)] * 4
            + [pltpu.VMEM((_BN, D), jnp.float32)] * 4
            + [pltpu.SemaphoreType.DMA] * 8
        ),
    )(_attn_sc)
    idx_flat = idx[:, :_KG].reshape(-1)
    return fn(q, k, v, idx_flat, maskf)


def _post_body(a_ref, x_ref, wo_ref, bo_ref, lw_ref, lb_ref, y_ref):
    dn = (((1,), (1,)), ((), ()))
    y = lax.dot_general(a_ref[...], wo_ref[...], dn,
                        preferred_element_type=jnp.float32)
    y = y + bo_ref[...] + x_ref[...]
    mu = jnp.mean(y, axis=1, keepdims=True)
    yc = y - mu
    var = jnp.mean(yc * yc, axis=1, keepdims=True)
    y_ref[...] = yc * lax.rsqrt(var + 1e-5) * lw_ref[...] + lb_ref[...]


def _post(attn, xp, Wo, bo, ln_w, ln_b):
    blk = 256
    grid = (NP // blk,)
    row = lambda i: (i, 0)
    full = lambda i: (0, 0)
    return pl.pallas_call(
        _post_body,
        grid=grid,
        in_specs=[
            pl.BlockSpec((blk, D), row),
            pl.BlockSpec((blk, D), row),
            pl.BlockSpec((D, D), full),
            pl.BlockSpec((1, D), full),
            pl.BlockSpec((1, D), full),
            pl.BlockSpec((1, D), full),
        ],
        out_specs=pl.BlockSpec((blk, D), row),
        out_shape=jax.ShapeDtypeStruct((NP, D), jnp.float32),
    )(attn, xp, Wo, bo.reshape(1, D), ln_w.reshape(1, D), ln_b.reshape(1, D))


def kernel(features, means, Wq, bq, Wk, bk, Wv, bv, Wo, bo, ln_w, ln_b):
    xp = jnp.pad(features, ((0, NP - N), (0, 0)))
    mrow = jnp.pad(means, ((0, NP - N), (0, 128 - 3)))
    mcolT = jnp.concatenate([means.T, jnp.zeros((5, N), jnp.float32)], axis=0)
    mcol = jnp.pad(mcolT, ((0, 0), (0, NP - N)), constant_values=COORD_PAD)

    q, k, v = _qkv(xp, Wq, bq, Wk, bk, Wv, bv)
    attns = []
    for c in range(_NSPLIT):
        lo = c * _HALF
        idxc, maskc = _knn(mrow[lo:lo + _HALF], mcol)
        attns.append(_attention(q[lo:lo + _HALF], k, v, idxc, maskc))
    attn = jnp.concatenate(attns, axis=0)
    y = _post(attn, xp, Wo, bo, ln_w, ln_b)
    return y[:N]
